# baseline (speedup 1.0000x reference)
import functools

import jax
import jax.numpy as jnp
from jax import lax
from jax.experimental import pallas as pl
from jax.experimental.pallas import tpu as pltpu
from jax.experimental.pallas import tpu_sc as plsc

DIM = 32
LANES = 16
WIDE = 128
PACK = WIDE // DIM
CHUNK = 128


def kernel(h, r, t, E, R):
    B = h.shape[0]
    V, D = E.shape
    mesh = plsc.VectorSubcoreMesh(core_axis_name="c", subcore_axis_name="s")
    NW = mesh.num_cores * mesh.num_subcores
    b_per_w = B // NW
    n_chunks = b_per_w // CHUNK

    E2 = E.reshape(V // PACK, WIDE)
    R2 = R.reshape(R.shape[0] // PACK, WIDE)

    def split(ix):
        q = (ix // PACK).reshape(NW, n_chunks, CHUNK)
        m = ((ix % PACK) * DIM).reshape(NW, n_chunks, CHUNK)
        return q, m

    hq, hm = split(h)
    rq, rm = split(r)
    tq, tm = split(t)

    idx_t = pltpu.VMEM((n_chunks, CHUNK), jnp.int32)
    buf_t = pltpu.VMEM((2, CHUNK, WIDE), jnp.float32)

    @functools.partial(
        pl.kernel,
        out_type=jax.ShapeDtypeStruct((B,), jnp.float32),
        mesh=mesh,
        scratch_types=[
            idx_t, idx_t, idx_t,
            idx_t, idx_t, idx_t,
            buf_t, buf_t, buf_t,
            pltpu.VMEM((b_per_w,), jnp.float32),
            pltpu.SemaphoreType.DMA,
        ],
        compiler_params=pltpu.CompilerParams(needs_layout_passes=False),
    )
    def transe(hq_hbm, hm_hbm, rq_hbm, rm_hbm, tq_hbm, tm_hbm, E_hbm, R_hbm,
               out_hbm, hq_v, rq_v, tq_v, hm_v, rm_v, tm_v,
               eh_v, rr_v, et_v, out_v, sem):
        wid = lax.axis_index("s") * mesh.num_cores + lax.axis_index("c")
        base = wid * b_per_w

        pltpu.sync_copy(hq_hbm.at[wid], hq_v)
        pltpu.sync_copy(rq_hbm.at[wid], rq_v)
        pltpu.sync_copy(tq_hbm.at[wid], tq_v)
        pltpu.sync_copy(hm_hbm.at[wid], hm_v)
        pltpu.sync_copy(rm_hbm.at[wid], rm_v)
        pltpu.sync_copy(tm_hbm.at[wid], tm_v)

        def fire(j):
            p = j % 2
            return [
                pltpu.async_copy(E_hbm.at[hq_v.at[j]], eh_v.at[p], sem),
                pltpu.async_copy(R_hbm.at[rq_v.at[j]], rr_v.at[p], sem),
                pltpu.async_copy(E_hbm.at[tq_v.at[j]], et_v.at[p], sem),
            ]

        lane = lax.iota(jnp.int32, LANES)
        pending = fire(0)
        for j in range(n_chunks):
            nxt = fire(j + 1) if j + 1 < n_chunks else []
            for c in pending:
                c.wait()
            pending = nxt
            p = j % 2
            ehb, rrb, etb = eh_v.at[p], rr_v.at[p], et_v.at[p]
            for g in range(CHUNK // LANES):
                rowi = g * LANES + lane
                ch0 = hm_v[j, pl.ds(g * LANES, LANES)]
                cr0 = rm_v[j, pl.ds(g * LANES, LANES)]
                ct0 = tm_v[j, pl.ds(g * LANES, LANES)]

                def body(d, carry):
                    acc, ch, cr, ct = carry
                    a = plsc.load_gather(ehb, [rowi, ch])
                    b = plsc.load_gather(rrb, [rowi, cr])
                    c = plsc.load_gather(etb, [rowi, ct])
                    acc = acc + jnp.abs(a + b - c)
                    return acc, ch + 1, cr + 1, ct + 1

                acc, _, _, _ = lax.fori_loop(
                    0, DIM, body,
                    (jnp.zeros((LANES,), jnp.float32), ch0, cr0, ct0))
                out_v[pl.ds(j * CHUNK + g * LANES, LANES)] = acc

        pltpu.sync_copy(out_v, out_hbm.at[pl.ds(base, b_per_w)])

    return transe(hq, hm, rq, rm, tq, tm, E2, R2)
